# baseline (speedup 1.0000x reference)
import jax
import jax.numpy as jnp
from jax import lax
from jax.experimental import pallas as pl
from jax.experimental.pallas import tpu as pltpu
from jax.experimental.pallas import tpu_sc as plsc

L = 16
NC, NS = 2, 16

B, C, H, W = 8, 384, 224, 224
CT, CL = C // 128, 128
WT, WS = W // 8, 8
NG = CL // L
NPAIR = B * CT
SC_PAIRS = 8
TC_PAIRS = NPAIR - SC_PAIRS
HQ = H // 4
K = 5
CH = 2
NCHUNK = HQ // CH
HB = 32
NHB = H // HB


def _insert(m, v):
    for k in range(K - 1):
        t = jnp.maximum(m[k], v)
        v = jnp.minimum(m[k], v)
        m[k] = t
    m[K - 1] = jnp.maximum(m[K - 1], v)


def _scan_body(x_hbm, cands_hbm, buf0, buf1, cand_v, sem0, sem1):
    wid = (lax.axis_index("c") * NS + lax.axis_index("s"))
    neg_inf = jnp.full((L,), -jnp.inf, dtype=jnp.float32)

    def scan_chunk(buf, st):
        def h_body(h, st):
            def wt_body(wtx, st):
                ms = [list(st[K * g:K * (g + 1)]) for g in range(NG)]
                for wsx in range(WS):
                    for g in range(NG):
                        v = buf[h, wtx, wsx, pl.ds(g * L, L)]
                        _insert(ms[g], v)
                return tuple(x for m in ms for x in m)

            return lax.fori_loop(0, WT, wt_body, st)

        return lax.fori_loop(0, CH, h_body, st)

    u = wid
    pair = u // 4
    q = u % 4
    b = pair // CT
    ct = pair % CT
    h0 = q * HQ

    pltpu.async_copy(x_hbm.at[b, pl.ds(h0, CH), :, ct], buf0, sem0)
    pltpu.async_copy(x_hbm.at[b, pl.ds(h0 + CH, CH), :, ct], buf1, sem1)

    def pair_body(g, st):
        hc = h0 + 2 * g * CH
        pltpu.make_async_copy(
            x_hbm.at[b, pl.ds(0, CH), :, ct], buf0, sem0).wait()
        st = scan_chunk(buf0, st)

        @pl.when(g < NCHUNK // 2 - 1)
        def _next0():
            pltpu.async_copy(
                x_hbm.at[b, pl.ds(hc + 2 * CH, CH), :, ct], buf0, sem0)

        pltpu.make_async_copy(
            x_hbm.at[b, pl.ds(0, CH), :, ct], buf1, sem1).wait()
        st = scan_chunk(buf1, st)

        @pl.when(g < NCHUNK // 2 - 1)
        def _next1():
            pltpu.async_copy(
                x_hbm.at[b, pl.ds(hc + 3 * CH, CH), :, ct], buf1, sem1)

        return st

    st = lax.fori_loop(0, NCHUNK // 2, pair_body, (neg_inf,) * (K * NG))

    for g in range(NG):
        for k5 in range(K):
            cand_v[k5, pl.ds(g * L, L)] = st[K * g + k5]
    pltpu.sync_copy(cand_v, cands_hbm.at[u])


def _merge_body(cands_hbm, out_hbm, merge_v, res_v):
    wid = lax.axis_index("c") * NS + lax.axis_index("s")

    @pl.when(wid < SC_PAIRS)
    def _merge():
        pltpu.sync_copy(cands_hbm.at[pl.ds(wid * 4, 4)], merge_v)
        for g in range(NG):
            m = [merge_v[0, k5, pl.ds(g * L, L)] for k5 in range(K)]
            for q in range(1, 4):
                for k5 in range(K):
                    _insert(m, merge_v[q, k5, pl.ds(g * L, L)])
            res_v[pl.ds(g * L, L)] = (m[0] + m[1] + m[2] + m[3] + m[4]) / 5.0
        pltpu.sync_copy(res_v, out_hbm.at[pl.ds(wid * CL, CL)])


def _tc_body(x_ref, out_ref, st_ref):
    j = pl.program_id(1)
    neg_inf = jnp.full((WS, CL), -jnp.inf, dtype=jnp.float32)

    @pl.when(j == 0)
    def _init():
        st_ref[...] = jnp.broadcast_to(neg_inf, (2, K, WS, CL))

    ms = [[st_ref[s, k] for k in range(K)] for s in range(2)]

    def h_body(h, st):
        def wt_body(wtx, st):
            m = [[st[s * K + k] for k in range(K)] for s in range(2)]
            _insert(m[0], x_ref[0, h, wtx, 0])
            _insert(m[1], x_ref[0, h + 1, wtx, 0])
            return tuple(x for mm in m for x in mm)

        return lax.fori_loop(0, WT, wt_body, st)

    st = lax.fori_loop(0, HB // 2, lambda h2, s: h_body(2 * h2, s),
                       tuple(x for mm in ms for x in mm),
                       unroll=False)
    for i in range(2 * K):
        st_ref[i // K, i % K] = st[i]

    @pl.when(j == NHB - 1)
    def _finalize():
        m = [st_ref[0, k] for k in range(K)]
        for k in range(K):
            _insert(m, st_ref[1, k])
        iota8 = lax.broadcasted_iota(jnp.int32, (WS, CL), 0)
        acc = jnp.zeros((1, CL), jnp.float32)
        neg = neg_inf
        for _ in range(K):
            g = jnp.max(m[0], axis=0, keepdims=True)
            eq = m[0] == g
            srow = jnp.min(jnp.where(eq, iota8, WS), axis=0, keepdims=True)
            sel = iota8 == srow
            acc = acc + g
            for k in range(K - 1):
                m[k] = jnp.where(sel, m[k + 1], m[k])
            m[K - 1] = jnp.where(sel, neg, m[K - 1])
        out_ref[...] = (acc / 5.0).reshape(1, 1, CL)


def _tc_pairs(xp):
    def in_map(p, j):
        pair = SC_PAIRS + p
        return (pair // CT, j, 0, pair % CT, 0, 0)

    return pl.pallas_call(
        _tc_body,
        grid=(TC_PAIRS, NHB),
        in_specs=[pl.BlockSpec((1, HB, WT, 1, WS, CL), in_map)],
        out_specs=pl.BlockSpec((1, 1, CL), lambda p, j: (p, 0, 0)),
        out_shape=jax.ShapeDtypeStruct((TC_PAIRS, 1, CL), jnp.float32),
        scratch_shapes=[pltpu.VMEM((2, K, WS, CL), jnp.float32)],
        compiler_params=pltpu.CompilerParams(
            dimension_semantics=("arbitrary", "arbitrary")),
    )(xp)


def kernel(x):
    xp = x.reshape(B, CT, CL, H, WT, WS).transpose(0, 3, 4, 1, 5, 2)
    mesh = plsc.VectorSubcoreMesh(core_axis_name="c", subcore_axis_name="s")
    cands = pl.kernel(
        _scan_body,
        out_type=jax.ShapeDtypeStruct((4 * SC_PAIRS, K, CL), jnp.float32),
        mesh=mesh,
        scratch_types=[
            pltpu.VMEM((CH, WT, WS, CL), jnp.float32),
            pltpu.VMEM((CH, WT, WS, CL), jnp.float32),
            pltpu.VMEM((K, CL), jnp.float32),
            pltpu.SemaphoreType.DMA,
            pltpu.SemaphoreType.DMA,
        ],
    )(xp)
    out_sc = pl.kernel(
        _merge_body,
        out_type=jax.ShapeDtypeStruct((SC_PAIRS * CL,), jnp.float32),
        mesh=mesh,
        scratch_types=[
            pltpu.VMEM((4, K, CL), jnp.float32),
            pltpu.VMEM((CL,), jnp.float32),
        ],
    )(cands)
    out_tc = _tc_pairs(xp)
    out = jnp.concatenate([out_sc, out_tc.reshape(TC_PAIRS * CL)])
    return out.reshape(B, C)


# speedup vs baseline: 2.1729x; 2.1729x over previous
import jax
import jax.numpy as jnp
from jax import lax
from jax.experimental import pallas as pl
from jax.experimental.pallas import tpu as pltpu
from jax.experimental.pallas import tpu_sc as plsc

L = 16
NC, NS = 2, 16

B, C, H, W = 8, 384, 224, 224
CT, CL = C // 128, 128
WT, WS = W // 8, 8
NG = CL // L
NPAIR = B * CT
SC_PAIRS = 12
TC_PAIRS = NPAIR - SC_PAIRS
QSPLIT = 8
TASKS_PER_TILE = SC_PAIRS * QSPLIT // (NC * NS)
HQ = H // QSPLIT
K = 5
CH = 2
NCHUNK = HQ // CH
HB = 32
NHB = H // HB


def _insert(m, v):
    for k in range(K - 1):
        t = jnp.maximum(m[k], v)
        v = jnp.minimum(m[k], v)
        m[k] = t
    m[K - 1] = jnp.maximum(m[K - 1], v)


def _scan_body(x_hbm, cands_hbm, buf0, buf1, cand_v, sem0, sem1):
    wid = lax.axis_index("c") * NS + lax.axis_index("s")
    neg_inf = jnp.full((L,), -jnp.inf, dtype=jnp.float32)

    def scan_chunk(buf, st):
        def h_body(h, st):
            def wt_body(wtx, st):
                ms = [list(st[K * g:K * (g + 1)]) for g in range(NG)]
                for wsx in range(WS):
                    for g in range(NG):
                        v = buf[h, wtx, wsx, pl.ds(g * L, L)]
                        _insert(ms[g], v)
                return tuple(x for m in ms for x in m)

            return lax.fori_loop(0, WT, wt_body, st)

        return lax.fori_loop(0, CH, h_body, st)

    def task_body(k, _):
        u = wid * TASKS_PER_TILE + k
        pair = u // QSPLIT
        q = u % QSPLIT
        b = pair // CT
        ct = pair % CT
        h0 = q * HQ

        pltpu.async_copy(x_hbm.at[b, pl.ds(h0, CH), :, ct], buf0, sem0)
        pltpu.async_copy(x_hbm.at[b, pl.ds(h0 + CH, CH), :, ct], buf1, sem1)

        def pair_body(g, st):
            hc = h0 + 2 * g * CH
            pltpu.make_async_copy(
                x_hbm.at[b, pl.ds(0, CH), :, ct], buf0, sem0).wait()
            st = scan_chunk(buf0, st)

            @pl.when(g < NCHUNK // 2 - 1)
            def _next0():
                pltpu.async_copy(
                    x_hbm.at[b, pl.ds(hc + 2 * CH, CH), :, ct], buf0, sem0)

            pltpu.make_async_copy(
                x_hbm.at[b, pl.ds(0, CH), :, ct], buf1, sem1).wait()
            st = scan_chunk(buf1, st)

            @pl.when(g < NCHUNK // 2 - 1)
            def _next1():
                pltpu.async_copy(
                    x_hbm.at[b, pl.ds(hc + 3 * CH, CH), :, ct], buf1, sem1)

            return st

        st = lax.fori_loop(0, NCHUNK // 2, pair_body, (neg_inf,) * (K * NG))

        for g in range(NG):
            for k5 in range(K):
                cand_v[k5, pl.ds(g * L, L)] = st[K * g + k5]
        pltpu.sync_copy(cand_v, cands_hbm.at[u])
        return 0

    lax.fori_loop(0, TASKS_PER_TILE, task_body, 0)


def _merge_body(cands_hbm, tc_dep_hbm, out_hbm, merge_v, res_v):
    del tc_dep_hbm
    wid = lax.axis_index("c") * NS + lax.axis_index("s")

    @pl.when(wid < SC_PAIRS)
    def _merge():
        pltpu.sync_copy(cands_hbm.at[pl.ds(wid * QSPLIT, QSPLIT)], merge_v)
        for g in range(NG):
            m = [merge_v[0, k5, pl.ds(g * L, L)] for k5 in range(K)]
            for q in range(1, QSPLIT):
                for k5 in range(K):
                    _insert(m, merge_v[q, k5, pl.ds(g * L, L)])
            res_v[pl.ds(g * L, L)] = (m[0] + m[1] + m[2] + m[3] + m[4]) / 5.0
        pltpu.sync_copy(res_v, out_hbm.at[pl.ds(wid * CL, CL)])


def _tc_body(x_ref, out_ref, st_ref):
    j = pl.program_id(1)
    neg_inf = jnp.full((WS, CL), -jnp.inf, dtype=jnp.float32)

    @pl.when(j == 0)
    def _init():
        st_ref[...] = jnp.broadcast_to(neg_inf, (4, K, WS, CL))

    ms = [[st_ref[s, k] for k in range(K)] for s in range(4)]

    def h_body(h, st):
        def wt_body(wtx2, st):
            m = [[st[s * K + k] for k in range(K)] for s in range(4)]
            wtx = 2 * wtx2
            _insert(m[0], x_ref[0, h, wtx, 0])
            _insert(m[1], x_ref[0, h + 1, wtx, 0])
            _insert(m[2], x_ref[0, h, wtx + 1, 0])
            _insert(m[3], x_ref[0, h + 1, wtx + 1, 0])
            return tuple(x for mm in m for x in mm)

        return lax.fori_loop(0, WT // 2, wt_body, st)

    st = lax.fori_loop(0, HB // 2, lambda h2, s: h_body(2 * h2, s),
                       tuple(x for mm in ms for x in mm))
    for i in range(4 * K):
        st_ref[i // K, i % K] = st[i]

    @pl.when(j == NHB - 1)
    def _finalize():
        m = [st_ref[0, k] for k in range(K)]
        for s in range(1, 4):
            for k in range(K):
                _insert(m, st_ref[s, k])
        iota8 = lax.broadcasted_iota(jnp.int32, (WS, CL), 0)
        acc = jnp.zeros((1, CL), jnp.float32)
        for _ in range(K):
            g = jnp.max(m[0], axis=0, keepdims=True)
            eq = m[0] == g
            srow = jnp.min(jnp.where(eq, iota8, WS), axis=0, keepdims=True)
            sel = iota8 == srow
            acc = acc + g
            for k in range(K - 1):
                m[k] = jnp.where(sel, m[k + 1], m[k])
            m[K - 1] = jnp.where(sel, neg_inf, m[K - 1])
        out_ref[...] = (acc / 5.0).reshape(1, 1, CL)


def _tc_pairs(xp):
    def in_map(p, j):
        pair = SC_PAIRS + p
        return (pair // CT, j, 0, pair % CT, 0, 0)

    return pl.pallas_call(
        _tc_body,
        grid=(TC_PAIRS, NHB),
        in_specs=[pl.BlockSpec((1, HB, WT, 1, WS, CL), in_map)],
        out_specs=pl.BlockSpec((1, 1, CL), lambda p, j: (p, 0, 0)),
        out_shape=jax.ShapeDtypeStruct((TC_PAIRS, 1, CL), jnp.float32),
        scratch_shapes=[pltpu.VMEM((4, K, WS, CL), jnp.float32)],
        compiler_params=pltpu.CompilerParams(
            dimension_semantics=("arbitrary", "arbitrary")),
    )(xp)


def kernel(x):
    xp = x.reshape(B, CT, CL, H, WT, WS).transpose(0, 3, 4, 1, 5, 2)
    mesh = plsc.VectorSubcoreMesh(core_axis_name="c", subcore_axis_name="s")
    cands = pl.kernel(
        _scan_body,
        out_type=jax.ShapeDtypeStruct((QSPLIT * SC_PAIRS, K, CL), jnp.float32),
        mesh=mesh,
        scratch_types=[
            pltpu.VMEM((CH, WT, WS, CL), jnp.float32),
            pltpu.VMEM((CH, WT, WS, CL), jnp.float32),
            pltpu.VMEM((K, CL), jnp.float32),
            pltpu.SemaphoreType.DMA,
            pltpu.SemaphoreType.DMA,
        ],
    )(xp)
    out_tc = _tc_pairs(xp)
    out_sc = pl.kernel(
        _merge_body,
        out_type=jax.ShapeDtypeStruct((SC_PAIRS * CL,), jnp.float32),
        mesh=mesh,
        scratch_types=[
            pltpu.VMEM((QSPLIT, K, CL), jnp.float32),
            pltpu.VMEM((CL,), jnp.float32),
        ],
    )(cands, out_tc)
    out = jnp.concatenate([out_sc, out_tc.reshape(TC_PAIRS * CL)])
    return out.reshape(B, C)
